# baseline (speedup 1.0000x reference)
import functools

import jax
import jax.numpy as jnp
from jax import lax
from jax.experimental import pallas as pl
from jax.experimental.pallas import tpu as pltpu
from jax.experimental.pallas import tpu_sc as plsc

NC = 2
NS = 16
CH = 128
ROWS_PER_SUB = 640

_mesh = plsc.VectorSubcoreMesh(core_axis_name="c", subcore_axis_name="s")


def _rsqrt16(x):
    i = plsc.bitcast(x, jnp.int32)
    y = plsc.bitcast(jnp.int32(0x5F3759DF) - (i >> 1), jnp.float32)
    for _ in range(3):
        y = y * (1.5 - 0.5 * x * y * y)
    return y


def _make_agg_kernel(nch, n_pad, dh, compute_dis):
    assert nch % 4 == 0 and nch >= 8
    out_type = [jax.ShapeDtypeStruct((NC, n_pad, dh), jnp.float32)]
    if compute_dis:
        out_type.append(jax.ShapeDtypeStruct((n_pad,), jnp.float32))

    @functools.partial(
        pl.kernel,
        out_type=out_type,
        mesh=_mesh,
        compiler_params=pltpu.CompilerParams(needs_layout_passes=False, use_tc_tiling_on_sc=False),
        scratch_types=[
            pltpu.VMEM((nch, CH), jnp.int32),
            pltpu.VMEM((nch, CH), jnp.int32),
            pltpu.VMEM((nch * CH,), jnp.float32),
            pltpu.VMEM((n_pad,), jnp.float32),
            pltpu.VMEM((dh,), jnp.float32),
            pltpu.VMEM((CH, dh), jnp.float32),
            pltpu.VMEM((CH, dh), jnp.float32),
            pltpu.VMEM((CH, dh), jnp.float32),
            pltpu.VMEM((CH, dh), jnp.float32),
            pltpu.SemaphoreType.DMA,
            pltpu.SemaphoreType.DMA,
            pltpu.SemaphoreType.DMA,
            pltpu.SemaphoreType.DMA,
            pltpu.SemaphoreType.DMA,
            pltpu.SemaphoreType.DMA,
            pltpu.SemaphoreType.DMA,
            pltpu.SemaphoreType.DMA,
            pltpu.VMEM_SHARED((n_pad, dh), jnp.float32),
            pltpu.VMEM_SHARED((n_pad, dh), jnp.float32),
            pltpu.VMEM_SHARED((n_pad,), jnp.float32),
        ],
    )
    def agg_kernel(*refs):
        if compute_dis:
            (xw_hbm, row_hbm, col_hbm, ew_hbm, b_hbm,
             h_out, dis_out,
             row_v, col_v, ew_v, dis_l, b_v, g0, g1, g2, g3,
             gs0, gs1, gs2, gs3, ss0, ss1, ss2, ss3,
             acc_sh, xw_sh, deg_sh) = refs
        else:
            (xw_hbm, row_hbm, col_hbm, ew_hbm, b_hbm, dis_hbm,
             h_out,
             row_v, col_v, ew_v, dis_l, b_v, g0, g1, g2, g3,
             gs0, gs1, gs2, gs3, ss0, ss1, ss2, ss3,
             acc_sh, xw_sh, deg_sh) = refs
        cid = lax.axis_index("c")
        sid = lax.axis_index("s")
        g = [g0, g1, g2, g3]
        gsem = [gs0, gs1, gs2, gs3]
        ssem = [ss0, ss1, ss2, ss3]

        def zb(r, _):
            for dd in range(dh // 16):
                g0[r, pl.ds(dd * 16, 16)] = jnp.zeros((16,), jnp.float32)
            return 0
        lax.fori_loop(0, CH, zb, 0)
        for b in range(ROWS_PER_SUB // CH):
            pltpu.sync_copy(
                g0, acc_sh.at[pl.ds(sid * ROWS_PER_SUB + b * CH, CH)])
        pltpu.sync_copy(row_hbm.at[sid], row_v)
        pltpu.sync_copy(col_hbm.at[sid], col_v)
        pltpu.sync_copy(ew_hbm.at[sid], ew_v)
        pltpu.sync_copy(b_hbm.at[cid], b_v)
        pltpu.sync_copy(xw_hbm.at[cid].at[pl.ds(sid * ROWS_PER_SUB, ROWS_PER_SUB)],
                        xw_sh.at[pl.ds(sid * ROWS_PER_SUB, ROWS_PER_SUB)])

        if compute_dis:
            def zd(t, _):
                dis_l[pl.ds(sid * ROWS_PER_SUB + t * 16, 16)] = (
                    jnp.zeros((16,), jnp.float32))
                return 0
            lax.fori_loop(0, ROWS_PER_SUB // 16, zd, 0)
            pltpu.sync_copy(dis_l.at[pl.ds(sid * ROWS_PER_SUB, ROWS_PER_SUB)],
                            deg_sh.at[pl.ds(sid * ROWS_PER_SUB, ROWS_PER_SUB)])
            plsc.subcore_barrier()

            def dbody(ch, _):
                pltpu.sync_copy(ew_v.at[pl.ds(ch * CH, CH)],
                                deg_sh.at[col_v.at[ch]], add=True)
                return 0
            lax.fori_loop(0, nch, dbody, 0)
            plsc.subcore_barrier()
            pltpu.sync_copy(deg_sh, dis_l)

            def nbody(t, _):
                x = dis_l[pl.ds(t * 16, 16)] + 1.0
                dis_l[pl.ds(t * 16, 16)] = _rsqrt16(x)
                return 0
            lax.fori_loop(0, n_pad // 16, nbody, 0)

            @pl.when(cid == 0)
            def _():
                pltpu.sync_copy(
                    dis_l.at[pl.ds(sid * ROWS_PER_SUB, ROWS_PER_SUB)],
                    dis_out.at[pl.ds(sid * ROWS_PER_SUB, ROWS_PER_SUB)])
            plsc.subcore_barrier()
        else:
            pltpu.sync_copy(dis_hbm, dis_l)
            plsc.subcore_barrier()

        def scale(gb, ch):
            def grp(k, _):
                ew16 = ew_v[pl.ds(ch * CH + k * 16, 16)]
                row16 = row_v[ch, pl.ds(k * 16, 16)]
                col16 = col_v[ch, pl.ds(k * 16, 16)]
                m16 = (ew16 * plsc.load_gather(dis_l, [row16])
                       * plsc.load_gather(dis_l, [col16]))
                for l in range(16):
                    j = k * 16 + l
                    mb = jnp.broadcast_to(m16[l], (16,))
                    for dd in range(dh // 16):
                        gb[j, pl.ds(dd * 16, 16)] = (
                            gb[j, pl.ds(dd * 16, 16)] * mb)
                return 0
            lax.fori_loop(0, CH // 16, grp, 0, unroll=4)

        pltpu.async_copy(xw_sh.at[row_v.at[0]], g[0], gsem[0])
        pltpu.async_copy(xw_sh.at[row_v.at[1]], g[1], gsem[1])

        def body(i, _):
            for b in range(4):
                ch = 4 * i + b
                b2 = (b + 2) % 4
                pltpu.make_async_copy(
                    xw_sh.at[row_v.at[ch]], g[b], gsem[b]).wait()
                scale(g[b], ch)
                pltpu.async_copy(g[b], acc_sh.at[col_v.at[ch]], ssem[b],
                                 add=True)

                @pl.when(ch >= 2)
                def _():
                    pltpu.make_async_copy(
                        g[b2], acc_sh.at[col_v.at[ch - 2]], ssem[b2]).wait()

                @pl.when(ch + 2 < nch)
                def _():
                    pltpu.async_copy(
                        xw_sh.at[row_v.at[ch + 2]], g[b2], gsem[b2])
            return 0
        lax.fori_loop(0, nch // 4, body, 0)

        for ch in (nch - 2, nch - 1):
            pltpu.make_async_copy(
                g[ch % 4], acc_sh.at[col_v.at[ch]], ssem[ch % 4]).wait()
        plsc.subcore_barrier()

        bvecs = [b_v[pl.ds(dd * 16, 16)] for dd in range(dh // 16)]
        for blk in range(ROWS_PER_SUB // CH):
            r0 = sid * ROWS_PER_SUB + blk * CH
            pltpu.sync_copy(acc_sh.at[pl.ds(r0, CH)], g0)
            pltpu.sync_copy(xw_sh.at[pl.ds(r0, CH)], g1)

            def rgrp(k, _):
                d16 = dis_l[pl.ds(r0 + k * 16, 16)]
                for l in range(16):
                    r = k * 16 + l
                    db = jnp.broadcast_to(d16[l], (16,))
                    db2 = db * db
                    for dd in range(dh // 16):
                        v = (g0[r, pl.ds(dd * 16, 16)]
                             + g1[r, pl.ds(dd * 16, 16)] * db2 + bvecs[dd])
                        g0[r, pl.ds(dd * 16, 16)] = jnp.maximum(v, 0.0)
                return 0
            lax.fori_loop(0, CH // 16, rgrp, 0, unroll=2)
            pltpu.sync_copy(g0, h_out.at[cid].at[pl.ds(r0, CH)])

    return agg_kernel


def _mm1_body(x_ref, w_ref, xs_ref):
    xw = jnp.dot(x_ref[...], w_ref[...], preferred_element_type=jnp.float32)
    dh = xw.shape[1] // 2
    xs_ref[0] = xw[:, :dh]
    xs_ref[1] = xw[:, dh:]


def _mm2_body(hp_ref, w2_ref, xs2_ref):
    h = jnp.concatenate([hp_ref[0], hp_ref[1]], axis=-1)
    xw2 = jnp.dot(h, w2_ref[...], preferred_element_type=jnp.float32)
    dh = xw2.shape[1] // 2
    xs2_ref[0] = xw2[:, :dh]
    xs2_ref[1] = xw2[:, dh:]


def kernel(x, edge_index, edge_weight, W1, b1, W2, b2):
    n, d_in = x.shape
    d = W1.shape[1]
    dh = d // 2
    e = edge_weight.shape[0]

    per_t = -(-e // NS)
    nch = -(-(-(-per_t // CH)) // 4) * 4
    e_pad = NS * nch * CH
    n_pad = NS * ROWS_PER_SUB

    row = edge_index[0].astype(jnp.int32)
    col = edge_index[1].astype(jnp.int32)
    ew = edge_weight.astype(jnp.float32)
    row_a = jnp.pad(row, (0, e_pad - e)).reshape(NS, nch, CH)
    col_a = jnp.pad(col, (0, e_pad - e)).reshape(NS, nch, CH)
    ew_a = jnp.pad(ew, (0, e_pad - e)).reshape(NS, nch * CH)

    R = 1000
    grid = (n // R,)
    half_spec = pl.BlockSpec((2, R, dh), lambda i: (0, i, 0))
    half_shape = jax.ShapeDtypeStruct((2, n_pad, dh), jnp.float32)

    xw1 = pl.pallas_call(
        _mm1_body,
        grid=grid,
        in_specs=[pl.BlockSpec((R, d_in), lambda i: (i, 0)),
                  pl.BlockSpec((d_in, d), lambda i: (0, 0))],
        out_specs=half_spec,
        out_shape=half_shape,
    )(x, W1)

    agg1 = _make_agg_kernel(nch, n_pad, dh, compute_dis=True)
    h1, dis = agg1(xw1, row_a, col_a, ew_a, b1.reshape(NC, dh))

    xw2 = pl.pallas_call(
        _mm2_body,
        grid=grid,
        in_specs=[half_spec, pl.BlockSpec((d, d), lambda i: (0, 0))],
        out_specs=half_spec,
        out_shape=half_shape,
    )(h1, W2)

    agg2 = _make_agg_kernel(nch, n_pad, dh, compute_dis=False)
    (h2,) = agg2(xw2, row_a, col_a, ew_a, b2.reshape(NC, dh), dis)

    return jnp.concatenate([h2[0, :n], h2[1, :n]], axis=-1)
